# baseline (speedup 1.0000x reference)
import functools

import jax
import jax.numpy as jnp
from jax.experimental import pallas as pl
from jax.experimental.pallas import tpu as pltpu

N_TOK = 16384
D_IN = 768
D_SAE = 24576
K = 32

BI = 512
BJ = 2048


def _enc_body(x_ref, w_ref, benc_ref, bdec_ref, out_ref):
    xb = x_ref[...] - bdec_ref[...]
    acc = jax.lax.dot_general(
        xb, w_ref[...], (((1,), (1,)), ((), ())),
        preferred_element_type=jnp.float32)
    out_ref[...] = jnp.maximum(acc + benc_ref[...], 0.0)


def _encode(x, W_enc, b_enc, b_dec):
    grid = (D_SAE // BJ, N_TOK // BI)
    return pl.pallas_call(
        _enc_body,
        grid=grid,
        in_specs=[
            pl.BlockSpec((BI, D_IN), lambda j, i: (i, 0)),
            pl.BlockSpec((BJ, D_IN), lambda j, i: (j, 0)),
            pl.BlockSpec((1, BJ), lambda j, i: (0, j)),
            pl.BlockSpec((1, D_IN), lambda j, i: (0, 0)),
        ],
        out_specs=pl.BlockSpec((BI, BJ), lambda j, i: (i, j)),
        out_shape=jax.ShapeDtypeStruct((N_TOK, D_SAE), jnp.float32),
        compiler_params=pltpu.CompilerParams(
            dimension_semantics=("arbitrary", "arbitrary"),
        ),
    )(x, W_enc, b_enc.reshape(1, D_SAE), b_dec.reshape(1, D_IN))


def kernel(x, W_enc, b_enc, W_dec, b_dec):
    latents = _encode(x, W_enc, b_enc, b_dec)
    _, top_indices = jax.lax.top_k(latents, K)
    top_acts = jnp.take_along_axis(latents, top_indices, axis=-1)
    dec_rows = jnp.take(W_dec, top_indices, axis=0)
    sae_out = jnp.einsum('nk,nkd->nd', top_acts, dec_rows) + b_dec
    e = sae_out - x
    total_variance = jnp.sum((x - jnp.mean(x, axis=0)) ** 2, axis=0)
    l2_loss = jnp.sum(e ** 2, axis=0)
    fvu = jnp.mean(l2_loss / total_variance)
    auxk_loss = jnp.asarray(0.0, dtype=sae_out.dtype)
    return (sae_out, top_acts, top_indices, fvu, auxk_loss)


# speedup vs baseline: 4.6374x; 4.6374x over previous
import functools

import jax
import jax.numpy as jnp
import numpy as np
from jax import lax
from jax.experimental import pallas as pl
from jax.experimental.pallas import tpu as pltpu
from jax.experimental.pallas import tpu_sc as plsc

N_TOK = 16384
D_IN = 768
D_SAE = 24576
K = 32
D16 = D_SAE // 16
NV16 = D16 // 16

NC = 2
NS = 16
NW = NC * NS
TPW = N_TOK // NW

BI = 512
BJ = 2048
NEG = np.float32(-3.0e38)


def _enc_body(x_ref, w_ref, benc_ref, bdec_ref, lat_ref, m16_ref, trep_ref,
              tmin_scr):
    j = pl.program_id(1)
    xb = x_ref[...] - bdec_ref[...]
    acc = lax.dot_general(xb, w_ref[...], (((1,), (1,)), ((), ())),
                          preferred_element_type=jnp.float32)
    latb = jnp.maximum(acc + benc_ref[...], 0.0)
    lat_ref[...] = latb
    lat3 = latb.reshape(BI, 16, BJ // 16)
    m16_ref[...] = jnp.max(lat3, axis=1)
    gmax = jnp.max(lat3, axis=-1)

    @pl.when(j == 0)
    def _():
        tmin_scr[...] = gmax

    @pl.when(j > 0)
    def _():
        tmin_scr[...] = jnp.minimum(tmin_scr[...], gmax)

    @pl.when(j == D_SAE // BJ - 1)
    def _():
        trep_ref[...] = jnp.broadcast_to(
            jnp.min(tmin_scr[...], axis=-1, keepdims=True), (BI, 16))


def _encode(x, W_enc, b_enc, b_dec):
    grid = (N_TOK // BI, D_SAE // BJ)
    return pl.pallas_call(
        _enc_body,
        grid=grid,
        in_specs=[
            pl.BlockSpec((BI, D_IN), lambda i, j: (i, 0)),
            pl.BlockSpec((BJ, D_IN), lambda i, j: (j, 0)),
            pl.BlockSpec((1, BJ), lambda i, j: (0, j)),
            pl.BlockSpec((1, D_IN), lambda i, j: (0, 0)),
        ],
        out_specs=[
            pl.BlockSpec((BI, BJ), lambda i, j: (i, j)),
            pl.BlockSpec((BI, BJ // 16), lambda i, j: (i, j)),
            pl.BlockSpec((BI, 16), lambda i, j: (i, 0)),
        ],
        out_shape=[
            jax.ShapeDtypeStruct((N_TOK, D_SAE), jnp.float32),
            jax.ShapeDtypeStruct((N_TOK, D16), jnp.float32),
            jax.ShapeDtypeStruct((N_TOK, 16), jnp.float32),
        ],
        scratch_shapes=[pltpu.VMEM((BI, 16), jnp.float32)],
        compiler_params=pltpu.CompilerParams(
            dimension_semantics=("arbitrary", "arbitrary"),
        ),
    )(x, W_enc, b_enc.reshape(1, D_SAE), b_dec.reshape(1, D_IN))


CCAP = 3072


def _sc_body(lat_hbm, m16_hbm, trep_hbm, wdec_hbm, bdec_hbm,
             acts_hbm, idx_hbm, out_hbm,
             lat_row, m16_v, trep_v, cidx_v, cval, cgid,
             acts_v, idxs_v, wrows, acc_v, bdec_v, tmpi, tmpf,
             v64, i64, sem0, sem1):
    wid = lax.axis_index("s") * NC + lax.axis_index("c")
    t0 = wid * TPW
    iota = lax.iota(jnp.int32, 16)
    zi = jnp.zeros((16,), jnp.int32)
    zf = jnp.zeros((16,), jnp.float32)

    pltpu.sync_copy(bdec_hbm, bdec_v)
    for i in range(NV16 + 1):
        cidx_v[pl.ds(i * 16, 16)] = zi

    def token_body(it, _carry):
        tok = t0 + it
        pltpu.sync_copy(lat_hbm.at[tok], lat_row)
        pltpu.sync_copy(m16_hbm.at[tok], m16_v)
        pltpu.sync_copy(trep_hbm.at[tok], trep_v)
        ts = trep_v[...]

        cnt = zi
        for i in range(NV16):
            v = m16_v[pl.ds(i * 16, 16)]
            msk = v >= ts
            cum = plsc.cumsum(msk.astype(jnp.int32))
            pos = cnt + cum - 1
            plsc.store_scatter(cidx_v, [pos], iota + (i * 16), mask=msk)
            tmpi[...] = cum
            cnt = cnt + plsc.load_gather(tmpi, [zi + 15])

        tmpi[...] = cnt
        cnt_s = tmpi[...][0]
        nb = (cnt_s + 15) // 16

        cv = zi
        def batch_body(b, cv):
            clv = cidx_v[pl.ds(b * 16, 16)]
            base16 = lax.shift_right_logical(clv, 7) * (BJ) + (clv & 127)
            valid = (iota + b * 16) < cnt
            for p in range(16):
                idxv = base16 + p * 128
                vv = plsc.load_gather(lat_row, [idxv])
                msk = (vv >= ts) & valid
                cum = plsc.cumsum(msk.astype(jnp.int32))
                pos = jnp.minimum(cv + cum - 1, CCAP - 1)
                plsc.store_scatter(cval, [pos], vv, mask=msk)
                plsc.store_scatter(cgid, [pos], idxv, mask=msk)
                tmpi[...] = cum
                cv = cv + plsc.load_gather(tmpi, [zi + 15])
            return cv
        cv = lax.fori_loop(0, nb, batch_body, zi)

        cv = jnp.minimum(cv, CCAP - 16)
        plsc.store_scatter(cval, [cv + iota], zf + NEG)
        plsc.store_scatter(cgid, [cv + iota], zi + (1 << 30))
        tmpi[...] = cv
        nv = (tmpi[...][0] + 15) // 16

        def merge2(a, ai, b, bi):
            rb = lax.rev(b, (0,))
            rbi = lax.rev(bi, (0,))
            m = a >= rb
            p = jnp.where(m, a, rb)
            pi = jnp.where(m, ai, rbi)
            q = jnp.where(m, rb, a)
            qi = jnp.where(m, rbi, ai)
            ps, pis = plsc.sort_key_val(p, pi, descending=True)
            qs, qis = plsc.sort_key_val(q, qi, descending=True)
            return ps, pis, qs, qis

        def sel_body(i, car):
            h0, h1, h2, h3, x0, x1, x2, x3 = car
            c = cval[pl.ds(i * 16, 16)]
            g = cgid[pl.ds(i * 16, 16)]
            cs, gs = plsc.sort_key_val(c, g, descending=True)
            h0, x0, cs, gs = merge2(h0, x0, cs, gs)
            h1, x1, cs, gs = merge2(h1, x1, cs, gs)
            h2, x2, cs, gs = merge2(h2, x2, cs, gs)
            h3, x3, cs, gs = merge2(h3, x3, cs, gs)
            return (h0, h1, h2, h3, x0, x1, x2, x3)

        init = (zf + NEG, zf + NEG, zf + NEG, zf + NEG,
                zi + (1 << 30), zi + (1 << 30), zi + (1 << 30),
                zi + (1 << 30))
        h0, h1, h2, h3, x0, x1, x2, x3 = lax.fori_loop(0, nv, sel_body, init)
        v64[pl.ds(0, 16)] = h0
        v64[pl.ds(16, 16)] = h1
        v64[pl.ds(32, 16)] = h2
        v64[pl.ds(48, 16)] = h3
        i64[pl.ds(0, 16)] = x0
        i64[pl.ds(16, 16)] = x1
        i64[pl.ds(32, 16)] = x2
        i64[pl.ds(48, 16)] = x3

        for ps in range(12):
            par = ps & 1
            for grp in range(2):
                lov = par + 2 * (iota + grp * 16)
                hiv = jnp.minimum(lov + 1, 63)
                va = plsc.load_gather(v64, [lov])
                vb = plsc.load_gather(v64, [hiv])
                ia = plsc.load_gather(i64, [lov])
                ib = plsc.load_gather(i64, [hiv])
                sw = (va == vb) & (ia > ib) & (lov < 63)
                plsc.store_scatter(i64, [lov], ib, mask=sw)
                plsc.store_scatter(i64, [hiv], ia, mask=sw)

        acts_v[pl.ds(0, 16)] = v64[pl.ds(0, 16)]
        acts_v[pl.ds(16, 16)] = v64[pl.ds(16, 16)]
        idxs_v[pl.ds(0, 16)] = i64[pl.ds(0, 16)]
        idxs_v[pl.ds(16, 16)] = i64[pl.ds(16, 16)]
        pltpu.sync_copy(acts_v, acts_hbm.at[tok])
        pltpu.sync_copy(idxs_v, idx_hbm.at[tok])

        pltpu.async_copy(wdec_hbm.at[idxs_v], wrows, sem0).wait()
        alo = acts_v[pl.ds(0, 16)]
        ahi = acts_v[pl.ds(16, 16)]
        asp = [jnp.full((16,), alo[k] if k < 16 else ahi[k - 16],
                        dtype=jnp.float32) for k in range(K)]

        def fma_body(jv, _):
            off = jv * 16
            a = bdec_v[pl.ds(off, 16)]
            for k in range(K):
                a = a + asp[k] * wrows[k, pl.ds(off, 16)]
            acc_v[pl.ds(off, 16)] = a
            return 0

        lax.fori_loop(0, D_IN // 16, fma_body, 0)
        pltpu.sync_copy(acc_v, out_hbm.at[tok])
        return 0

    lax.fori_loop(0, TPW, token_body, 0)


def _sc_topk_decode(lat, m16, trep, W_dec, b_dec):
    mesh = plsc.VectorSubcoreMesh(core_axis_name="c", subcore_axis_name="s",
                                  num_cores=NC, num_subcores=NS)
    f = functools.partial(
        pl.kernel,
        out_type=[
            jax.ShapeDtypeStruct((N_TOK, K), jnp.float32),
            jax.ShapeDtypeStruct((N_TOK, K), jnp.int32),
            jax.ShapeDtypeStruct((N_TOK, D_IN), jnp.float32),
        ],
        mesh=mesh,
        scratch_types=[
            pltpu.VMEM((D_SAE,), jnp.float32),
            pltpu.VMEM((D16,), jnp.float32),
            pltpu.VMEM((16,), jnp.float32),
            pltpu.VMEM((D16 + 16,), jnp.int32),
            pltpu.VMEM((CCAP,), jnp.float32),
            pltpu.VMEM((CCAP,), jnp.int32),
            pltpu.VMEM((K,), jnp.float32),
            pltpu.VMEM((K,), jnp.int32),
            pltpu.VMEM((K, D_IN), jnp.float32),
            pltpu.VMEM((D_IN,), jnp.float32),
            pltpu.VMEM((D_IN,), jnp.float32),
            pltpu.VMEM((16,), jnp.int32),
            pltpu.VMEM((16,), jnp.float32),
            pltpu.VMEM((64,), jnp.float32),
            pltpu.VMEM((64,), jnp.int32),
            pltpu.SemaphoreType.DMA,
            pltpu.SemaphoreType.DMA,
        ],
        compiler_params=pltpu.CompilerParams(needs_layout_passes=False),
    )(_sc_body)
    return f(lat, m16, trep, W_dec, b_dec)


BC = 512


def _fvu_body(x_ref, s_ref, fvu_ref, sx, sx2, se2):
    i = pl.program_id(0)

    @pl.when(i == 0)
    def _():
        sx[...] = jnp.zeros_like(sx)
        sx2[...] = jnp.zeros_like(sx2)
        se2[...] = jnp.zeros_like(se2)

    xb = x_ref[...]
    e = s_ref[...] - xb
    sx[...] += jnp.sum(xb, axis=0, keepdims=True)
    sx2[...] += jnp.sum(xb * xb, axis=0, keepdims=True)
    se2[...] += jnp.sum(e * e, axis=0, keepdims=True)

    @pl.when(i == N_TOK // BC - 1)
    def _():
        tv = sx2[...] - sx[...] * sx[...] * (1.0 / N_TOK)
        fvu_ref[...] = jnp.mean(se2[...] / tv).reshape(1, 1)


def _fvu(x, sae_out):
    return pl.pallas_call(
        _fvu_body,
        grid=(N_TOK // BC,),
        in_specs=[
            pl.BlockSpec((BC, D_IN), lambda i: (i, 0)),
            pl.BlockSpec((BC, D_IN), lambda i: (i, 0)),
        ],
        out_specs=pl.BlockSpec((1, 1), lambda i: (0, 0)),
        out_shape=jax.ShapeDtypeStruct((1, 1), jnp.float32),
        scratch_shapes=[
            pltpu.VMEM((1, D_IN), jnp.float32),
            pltpu.VMEM((1, D_IN), jnp.float32),
            pltpu.VMEM((1, D_IN), jnp.float32),
        ],
        compiler_params=pltpu.CompilerParams(
            dimension_semantics=("arbitrary",),
        ),
    )(x, sae_out)


def kernel(x, W_enc, b_enc, W_dec, b_dec):
    lat, m16, trep = _encode(x, W_enc, b_enc, b_dec)
    top_acts, top_idx, sae_out = _sc_topk_decode(lat, m16, trep, W_dec, b_dec)
    fvu = _fvu(x, sae_out).reshape(())
    auxk_loss = jnp.zeros((), dtype=jnp.float32)
    return (sae_out, top_acts, top_idx, fvu, auxk_loss)
